# baseline (speedup 1.0000x reference)
import jax
import jax.numpy as jnp
from jax import lax
from jax.experimental import pallas as pl
from jax.experimental.pallas import tpu as pltpu

_NUM_PID = 15080
_SCALAR = 10.0
_ROWS = 64
_D = 2048
_C = 1024
_NFULL = _NUM_PID // _C
_TAIL = _NUM_PID - _NFULL * _C
_NBUF = 4
_TPAD = -(-_TAIL // 128) * 128


def _oim_kernel(ids_ref, feat_ref, lut_ref, out_ref,
                b0, b1, b2, b3, bt, m0, m1, m2, m3, mt):
    bufs = [b0, b1, b2, b3]
    sems = [m0, m1, m2, m3]

    def _copy(chunk):
        return pltpu.make_async_copy(
            lut_ref.at[pl.ds(chunk * _C, _C)], bufs[chunk % _NBUF],
            sems[chunk % _NBUF])

    for c in range(_NBUF):
        _copy(c).start()
    feat = feat_ref[...]
    pids = ids_ref[:, :1]
    row_ok = pids > -1
    safe = jnp.where(row_ok, pids, 0)

    s = jnp.zeros((_ROWS, 128), jnp.float32)
    p = jnp.zeros((_ROWS, 128), jnp.float32)

    for j in range(_NFULL):
        _copy(j).wait()
        block = bufs[j % _NBUF][...]
        logits = _SCALAR * lax.dot_general(
            feat, block, (((1,), (1,)), ((), ())),
            preferred_element_type=jnp.float32)
        col = j * _C + lax.broadcasted_iota(jnp.int32, (_ROWS, _C), 1)
        s = s + jnp.exp(logits - _SCALAR).reshape(_ROWS, _C // 128, 128).sum(axis=1)
        p = p + jnp.where(col == safe, logits, 0.0).reshape(
            _ROWS, _C // 128, 128).sum(axis=1)
        nxt = j + _NBUF
        if nxt < _NFULL:
            _copy(nxt).start()
        elif nxt == _NFULL:
            pltpu.make_async_copy(
                lut_ref.at[pl.ds(_NFULL * _C, _TAIL)], bt, mt).start()

    pltpu.make_async_copy(
        lut_ref.at[pl.ds(_NFULL * _C, _TAIL)], bt, mt).wait()
    logits = _SCALAR * lax.dot_general(
        feat, bt[...], (((1,), (1,)), ((), ())),
        preferred_element_type=jnp.float32)
    pad = -jnp.inf * jnp.ones((_ROWS, _TPAD - _TAIL), jnp.float32)
    logits = jnp.concatenate([logits, pad], axis=1)
    col = _NFULL * _C + lax.broadcasted_iota(jnp.int32, (_ROWS, _TPAD), 1)
    e = jnp.where(col < _NUM_PID, jnp.exp(logits - _SCALAR), 0.0)
    s = s + e.reshape(_ROWS, _TPAD // 128, 128).sum(axis=1)
    p = p + jnp.where(col == safe, logits, 0.0).reshape(_ROWS, _TPAD // 128, 128).sum(axis=1)

    s_tot = s.sum(axis=1, keepdims=True)
    p_tot = p.sum(axis=1, keepdims=True)
    lse = jnp.log(s_tot) + _SCALAR
    per_row = jnp.where(row_ok, lse - p_tot, 0.0)
    cnt = jnp.sum(row_ok.astype(jnp.float32))
    out_ref[0, 0] = jnp.sum(per_row) / cnt


def kernel(features, gt_labels, lut):
    pids = gt_labels.reshape(-1, gt_labels.shape[-1])[:, -1].astype(jnp.int32)
    ids2d = jnp.broadcast_to(pids[:, None], (_ROWS, 128))
    loss = pl.pallas_call(
        _oim_kernel,
        in_specs=[
            pl.BlockSpec((_ROWS, 128), lambda: (0, 0)),
            pl.BlockSpec((_ROWS, _D), lambda: (0, 0)),
            pl.BlockSpec(memory_space=pl.ANY),
        ],
        out_specs=pl.BlockSpec(memory_space=pltpu.SMEM),
        out_shape=jax.ShapeDtypeStruct((1, 1), jnp.float32),
        scratch_shapes=[
            pltpu.VMEM((_C, _D), jnp.float32),
            pltpu.VMEM((_C, _D), jnp.float32),
            pltpu.VMEM((_C, _D), jnp.float32),
            pltpu.VMEM((_C, _D), jnp.float32),
            pltpu.VMEM((_TAIL, _D), jnp.float32),
            pltpu.SemaphoreType.DMA,
            pltpu.SemaphoreType.DMA,
            pltpu.SemaphoreType.DMA,
            pltpu.SemaphoreType.DMA,
            pltpu.SemaphoreType.DMA,
        ],
    )(ids2d, features, lut)
    return loss[0, 0]


# speedup vs baseline: 1.0619x; 1.0619x over previous
import jax
import jax.numpy as jnp
from jax.experimental import pallas as pl
from jax.experimental.pallas import tpu as pltpu

_NUM_PID = 15080
_SCALAR = 10.0
_ROWS = 64
_CHUNK = 1408


def _oim_kernel(ids_ref, feat_ref, lut_ref, out_ref, s_ref, p_ref):
    j = pl.program_id(0)
    nc = pl.num_programs(0)

    @pl.when(j == 0)
    def _init():
        s_ref[...] = jnp.zeros_like(s_ref)
        p_ref[...] = jnp.zeros_like(p_ref)

    logits = _SCALAR * jax.lax.dot_general(
        feat_ref[...], lut_ref[...], (((1,), (1,)), ((), ())),
        preferred_element_type=jnp.float32)

    base = j * _CHUNK
    col = base + jax.lax.broadcasted_iota(jnp.int32, (_ROWS, _CHUNK), 1)

    e = jnp.where(col < _NUM_PID, jnp.exp(logits - _SCALAR), 0.0)
    s_ref[...] += e.reshape(_ROWS, _CHUNK // 128, 128).sum(axis=1)

    pids = ids_ref[:, :1]
    row_ok = pids > -1
    safe = jnp.where(row_ok, pids, 0)
    pick = jnp.where(col == safe, logits, 0.0)
    p_ref[...] += pick.reshape(_ROWS, _CHUNK // 128, 128).sum(axis=1)

    @pl.when(j == nc - 1)
    def _fin():
        s_tot = s_ref[...].sum(axis=1, keepdims=True)
        p_tot = p_ref[...].sum(axis=1, keepdims=True)
        lse = jnp.log(s_tot) + _SCALAR
        per_row = jnp.where(row_ok, lse - p_tot, 0.0)
        cnt = jnp.sum(row_ok.astype(jnp.float32))
        out_ref[0, 0] = jnp.sum(per_row) / cnt


def kernel(features, gt_labels, lut):
    pids = gt_labels.reshape(-1, gt_labels.shape[-1])[:, -1].astype(jnp.int32)
    ids2d = jnp.broadcast_to(pids[:, None], (_ROWS, 128))
    nc = pl.cdiv(_NUM_PID, _CHUNK)
    loss = pl.pallas_call(
        _oim_kernel,
        grid=(nc,),
        in_specs=[
            pl.BlockSpec((_ROWS, 128), lambda j: (0, 0)),
            pl.BlockSpec((_ROWS, features.shape[1]), lambda j: (0, 0)),
            pl.BlockSpec((_CHUNK, lut.shape[1]), lambda j: (j, 0)),
        ],
        out_specs=pl.BlockSpec(memory_space=pltpu.SMEM),
        out_shape=jax.ShapeDtypeStruct((1, 1), jnp.float32),
        scratch_shapes=[
            pltpu.VMEM((_ROWS, 128), jnp.float32),
            pltpu.VMEM((_ROWS, 128), jnp.float32),
        ],
    )(ids2d, features, lut)
    return loss[0, 0]
